# baseline (speedup 1.0000x reference)
import functools

import jax
import jax.numpy as jnp
from jax import lax
from jax.experimental import pallas as pl
from jax.experimental.pallas import tpu as pltpu
from jax.experimental.pallas import tpu_sc as plsc

N = 100000
E = 3200000
G = 256
HEADS = 8
DH = 16
EPS = 1e-5

NC = 2
NS = 16
NW = NC * NS
RPT = 1568
ER2 = NS * RPT
EPAD = ER2 * 128
KB = 16
NBLK = RPT // KB
KB2 = 8
HALF = 50048
ACC_ROWS = 50176
NPT = ACC_ROWS // NS
ZCH = 392
NZCH = NPT // ZCH
PR2 = 1024
PN = PR2 * 128
PACC_ROWS = 512
PRPT = PR2 // NW

_MESH = plsc.VectorSubcoreMesh(core_axis_name="c", subcore_axis_name="s")


def _vgather(v, idx):
    return lax.gather(
        v, idx[:, None],
        dimension_numbers=lax.GatherDimensionNumbers(
            offset_dims=(), collapsed_slice_dims=(0,), start_index_map=(0,)),
        slice_sizes=(1,),
        mode=lax.GatherScatterMode.PROMISE_IN_BOUNDS)


def _zero_acc(acc, zbuf):
    def zb(i, _):
        zbuf[i, :] = jnp.zeros((16,), jnp.float32)
        return 0
    lax.fori_loop(0, ZCH, zb, 0)
    s = lax.axis_index("s")
    for k in range(NZCH):
        pltpu.sync_copy(zbuf, acc.at[pl.ds(s * NPT + k * ZCH, ZCH)])


def _localize(didx, lidx, kb):
    cbase = lax.axis_index("c") * HALF
    lanes = lax.iota(jnp.int32, 16)

    def tr(i, _):
        for j in range(8):
            d = didx[i, j * 16:(j + 1) * 16]
            loc = d - cbase
            ok = (loc >= 0) & (loc < HALF)
            lidx[i, j * 16:(j + 1) * 16] = jnp.where(ok, loc, HALF + lanes)
        return 0

    lax.fori_loop(0, kb, tr, 0)


def _writeout(acc, out):
    s = lax.axis_index("s")
    c = lax.axis_index("c")
    pltpu.sync_copy(acc.at[pl.ds(s * NPT, NPT)],
                    out.at[c, pl.ds(s * NPT, NPT)])


@functools.partial(
    pl.kernel,
    out_type=jax.ShapeDtypeStruct((NC, ACC_ROWS, 16), jnp.float32),
    mesh=_MESH,
    compiler_params=pltpu.CompilerParams(use_tc_tiling_on_sc=False),
    scratch_types=[
        pltpu.VMEM((KB, 128), jnp.int32),
        pltpu.VMEM((KB, 128), jnp.int32),
        pltpu.VMEM((KB * 128, 16), jnp.float32),
        pltpu.VMEM((ZCH, 16), jnp.float32),
        pltpu.VMEM_SHARED((ACC_ROWS, 16), jnp.float32),
        pltpu.SemaphoreType.DMA,
    ],
)
def _sc_seg16(table, src2, dst2, out, sidx, didx, rows, zbuf, acc, sem):
    _zero_acc(acc, zbuf)
    plsc.subcore_barrier()
    base = lax.axis_index("s") * RPT

    def blk(b, _):
        rb = base + b * KB
        pltpu.sync_copy(src2.at[pl.ds(rb, KB)], sidx)
        pltpu.sync_copy(dst2.at[pl.ds(rb, KB)], didx)
        _localize(didx, didx, KB)
        descs = [
            pltpu.async_copy(table.at[sidx.at[j]],
                             rows.at[pl.ds(j * 128, 128)], sem)
            for j in range(KB)
        ]
        for d in descs:
            d.wait()
        for j in range(KB):
            pltpu.sync_copy(rows.at[pl.ds(j * 128, 128)],
                            acc.at[didx.at[j]], add=True)
        return 0

    lax.fori_loop(0, NBLK, blk, 0)
    plsc.subcore_barrier()
    _writeout(acc, out)


@functools.partial(
    pl.kernel,
    out_type=jax.ShapeDtypeStruct((NC, ACC_ROWS, 16), jnp.float32),
    mesh=_MESH,
    compiler_params=pltpu.CompilerParams(use_tc_tiling_on_sc=False),
    scratch_types=[
        pltpu.VMEM((KB, 128), jnp.int32),
        pltpu.VMEM((128, 16), jnp.float32),
        pltpu.VMEM((ZCH, 16), jnp.float32),
        pltpu.VMEM_SHARED((ACC_ROWS, 16), jnp.float32),
    ],
)
def _sc_deg(dst2, out, didx, ones, zbuf, acc):
    _zero_acc(acc, zbuf)
    def ob(i, _):
        ones[i, :] = jnp.full((16,), 1.0, jnp.float32)
        return 0
    lax.fori_loop(0, 128, ob, 0)
    plsc.subcore_barrier()
    base = lax.axis_index("s") * RPT

    def blk(b, _):
        rb = base + b * KB
        pltpu.sync_copy(dst2.at[pl.ds(rb, KB)], didx)
        _localize(didx, didx, KB)
        for j in range(KB):
            pltpu.sync_copy(ones, acc.at[didx.at[j]], add=True)
        return 0

    lax.fori_loop(0, NBLK, blk, 0)
    plsc.subcore_barrier()
    _writeout(acc, out)


@functools.partial(
    pl.kernel,
    out_type=jax.ShapeDtypeStruct((NC, ACC_ROWS, 16), jnp.float32),
    mesh=_MESH,
    compiler_params=pltpu.CompilerParams(use_tc_tiling_on_sc=False),
    scratch_types=[
        pltpu.VMEM((KB, 128), jnp.int32),
        pltpu.VMEM((KB, 128), jnp.int32),
        pltpu.VMEM((KB, 128), jnp.int32),
        pltpu.VMEM((KB * 128, 16), jnp.float32),
        pltpu.VMEM((KB * 128, 16), jnp.float32),
        pltpu.VMEM((16,), jnp.float32),
        pltpu.VMEM((ZCH, 16), jnp.float32),
        pltpu.VMEM_SHARED((ACC_ROWS, 16), jnp.float32),
        pltpu.SemaphoreType.DMA,
    ],
)
def _sc_gat_den(ast, adt, mvec, src2, dst2, out,
                sidx, didx, lidx, rs, rd, mbuf, zbuf, acc, sem):
    _zero_acc(acc, zbuf)
    pltpu.sync_copy(mvec, mbuf)
    plsc.subcore_barrier()
    base = lax.axis_index("s") * RPT

    def blk(b, _):
        rb = base + b * KB
        pltpu.sync_copy(src2.at[pl.ds(rb, KB)], sidx)
        pltpu.sync_copy(dst2.at[pl.ds(rb, KB)], didx)
        _localize(didx, lidx, KB)
        descs = [
            pltpu.async_copy(ast.at[sidx.at[j]],
                             rs.at[pl.ds(j * 128, 128)], sem)
            for j in range(KB)
        ] + [
            pltpu.async_copy(adt.at[didx.at[j]],
                             rd.at[pl.ds(j * 128, 128)], sem)
            for j in range(KB)
        ]
        for d in descs:
            d.wait()
        m = mbuf[:]

        def ed(e, _):
            l = rs[e, :] + rd[e, :]
            l = jnp.maximum(l, l * 0.2)
            rs[e, :] = jnp.exp(l - m)
            return 0

        lax.fori_loop(0, KB * 128, ed, 0)
        for j in range(KB):
            pltpu.sync_copy(rs.at[pl.ds(j * 128, 128)],
                            acc.at[lidx.at[j]], add=True)
        return 0

    lax.fori_loop(0, NBLK, blk, 0)
    plsc.subcore_barrier()
    _writeout(acc, out)


@functools.partial(
    pl.kernel,
    out_type=jax.ShapeDtypeStruct((NC, ACC_ROWS, 16), jnp.float32),
    mesh=_MESH,
    compiler_params=pltpu.CompilerParams(use_tc_tiling_on_sc=False),
    scratch_types=[
        pltpu.VMEM((KB2, 128), jnp.int32),
        pltpu.VMEM((KB2, 128), jnp.int32),
        pltpu.VMEM((KB2, 128), jnp.int32),
        pltpu.VMEM((128, 16), jnp.float32),
        pltpu.VMEM((128, 16), jnp.float32),
        pltpu.VMEM((128, 128), jnp.float32),
        pltpu.VMEM((128, 16), jnp.float32),
        pltpu.VMEM((16,), jnp.float32),
        pltpu.VMEM((ZCH, 16), jnp.float32),
        pltpu.VMEM_SHARED((ACC_ROWS, 16), jnp.float32),
        pltpu.SemaphoreType.DMA,
    ],
)
def _sc_gat_num(ast, adit, htab, mvec, src2, dst2, out,
                sidx, didx, lidx, rs, rdi, rh, contrib, mbuf, zbuf, acc, sem):
    _zero_acc(acc, zbuf)
    pltpu.sync_copy(mvec, mbuf)
    plsc.subcore_barrier()
    base = lax.axis_index("s") * RPT
    perm = (lax.iota(jnp.int32, 16) % 8) + 8

    def blk(b, _):
        rb = base + b * KB2
        pltpu.sync_copy(src2.at[pl.ds(rb, KB2)], sidx)
        pltpu.sync_copy(dst2.at[pl.ds(rb, KB2)], didx)
        _localize(didx, lidx, KB2)
        for j in range(KB2):
            descs = [
                pltpu.async_copy(ast.at[sidx.at[j]], rs, sem),
                pltpu.async_copy(adit.at[didx.at[j]], rdi, sem),
                pltpu.async_copy(htab.at[sidx.at[j]], rh, sem),
            ]
            for d in descs:
                d.wait()
            m = mbuf[:]

            def ed(e, _):
                di = rdi[e, :]
                l = rs[e, :] + di
                l = jnp.maximum(l, l * 0.2)
                ex = jnp.exp(l - m)
                alpha = ex * _vgather(di, perm)
                acc16 = jnp.zeros((16,), jnp.float32)
                for h in range(HEADS):
                    sp = _vgather(alpha, jnp.full((16,), h, jnp.int32))
                    acc16 = acc16 + sp * rh[e, h * 16:(h + 1) * 16]
                contrib[e, :] = acc16
                return 0

            lax.fori_loop(0, 128, ed, 0)
            pltpu.sync_copy(contrib, acc.at[lidx.at[j]], add=True)
        return 0

    lax.fori_loop(0, RPT // KB2, blk, 0)
    plsc.subcore_barrier()
    _writeout(acc, out)


@functools.partial(
    pl.kernel,
    out_type=jax.ShapeDtypeStruct((NC, G, 16), jnp.float32),
    mesh=_MESH,
    compiler_params=pltpu.CompilerParams(use_tc_tiling_on_sc=False),
    scratch_types=[
        pltpu.VMEM((PRPT, 128), jnp.int32),
        pltpu.VMEM((128, 16), jnp.float32),
        pltpu.VMEM((PACC_ROWS // NS, 16), jnp.float32),
        pltpu.VMEM_SHARED((PACC_ROWS, 16), jnp.float32),
    ],
)
def _sc_pool(vals, b2, out, bidx, vbuf, zbuf, acc):
    s = lax.axis_index("s")
    c = lax.axis_index("c")
    def zb(i, _):
        zbuf[i, :] = jnp.zeros((16,), jnp.float32)
        return 0
    lax.fori_loop(0, PACC_ROWS // NS, zb, 0)
    pltpu.sync_copy(zbuf, acc.at[pl.ds(s * (PACC_ROWS // NS),
                                       PACC_ROWS // NS)])
    plsc.subcore_barrier()
    base = (s * NC + c) * PRPT
    pltpu.sync_copy(b2.at[pl.ds(base, PRPT)], bidx)

    def row(r, _):
        pltpu.sync_copy(vals.at[pl.ds((base + r) * 128, 128)], vbuf)
        pltpu.sync_copy(vbuf, acc.at[bidx.at[r]], add=True)
        return 0

    lax.fori_loop(0, PRPT, row, 0)
    plsc.subcore_barrier()
    pltpu.sync_copy(acc.at[pl.ds(s * (G // NS), G // NS)],
                    out.at[c, pl.ds(s * (G // NS), G // NS)])


def _head_kernel(p0_ref, p1_ref, w_ref, b_ref, o_ref):
    o_ref[...] = (
        jnp.dot(p0_ref[...] + p1_ref[...], w_ref[...],
                preferred_element_type=jnp.float32)
        + b_ref[...]
    )


BLK = 2000
NGRID = N // BLK


def _nb(d):
    return pl.BlockSpec((BLK, d), lambda i: (i, 0))


def _fixed(shape):
    return pl.BlockSpec(shape, lambda i: tuple(0 for _ in shape))


def _tc_dis_kernel(d_ref, o_ref):
    o_ref[...] = lax.rsqrt(d_ref[...] + 1.0)


def _tc_dis(d):
    return pl.pallas_call(
        _tc_dis_kernel,
        grid=(NGRID,),
        in_specs=[_nb(1)],
        out_specs=_nb(1),
        out_shape=jax.ShapeDtypeStruct((N, 1), jnp.float32),
    )(d)


def _tc_mm_scale_kernel(x_ref, w_ref, dis_ref, o_ref):
    o_ref[...] = (
        jnp.dot(x_ref[...], w_ref[...], preferred_element_type=jnp.float32)
        * dis_ref[...]
    )


def _tc_mm_scale(x, W, dis):
    din, dout = W.shape
    return pl.pallas_call(
        _tc_mm_scale_kernel,
        grid=(NGRID,),
        in_specs=[_nb(din), _fixed((din, dout)), _nb(1)],
        out_specs=_nb(dout),
        out_shape=jax.ShapeDtypeStruct((N, dout), jnp.float32),
    )(x, W, dis)


def _tc_stats_kernel(p_ref, u_ref, dis_ref, o_ref):
    y = (p_ref[...] + u_ref[...]) * dis_ref[...]

    @pl.when(pl.program_id(0) == 0)
    def _():
        o_ref[...] = jnp.zeros_like(o_ref)

    o_ref[0, :] += jnp.sum(y, axis=0)
    o_ref[1, :] += jnp.sum(y * y, axis=0)


def _tc_stats(p, u, dis):
    dout = u.shape[1]
    return pl.pallas_call(
        _tc_stats_kernel,
        grid=(NGRID,),
        in_specs=[_nb(dout), _nb(dout), _nb(1)],
        out_specs=_fixed((2, dout)),
        out_shape=jax.ShapeDtypeStruct((2, dout), jnp.float32),
    )(p, u, dis)


def _tc_bn_kernel(p_ref, u_ref, dis_ref, st_ref, g_ref, be_ref, o_ref):
    y = (p_ref[...] + u_ref[...]) * dis_ref[...]
    mu = st_ref[0, :] / N
    var = st_ref[1, :] / N - mu * mu
    xn = g_ref[...] * (y - mu[None, :]) * lax.rsqrt(var[None, :] + EPS)
    o_ref[...] = jax.nn.relu(xn + be_ref[...])


def _tc_bn_relu(p, u, dis, st, gm, be):
    dout = u.shape[1]
    return pl.pallas_call(
        _tc_bn_kernel,
        grid=(NGRID,),
        in_specs=[_nb(dout), _nb(dout), _nb(1), _fixed((2, dout)),
                  _fixed((1, dout)), _fixed((1, dout))],
        out_specs=_nb(dout),
        out_shape=jax.ShapeDtypeStruct((N, dout), jnp.float32),
    )(p, u, dis, st, gm.reshape(1, -1), be.reshape(1, -1))


def _tc_gat_prep_kernel(h_ref, wg_ref, as_ref, ad_ref,
                        ht_ref, ast_ref, adt_ref, mx_ref):
    ht = jnp.dot(h_ref[...], wg_ref[...], preferred_element_type=jnp.float32)
    as8 = jnp.dot(ht, as_ref[...], preferred_element_type=jnp.float32)
    ad8 = jnp.dot(ht, ad_ref[...], preferred_element_type=jnp.float32)
    ht_ref[...] = ht
    ast_ref[...] = jnp.concatenate([as8, as8], axis=1)
    adt_ref[...] = jnp.concatenate([ad8, ad8], axis=1)

    @pl.when(pl.program_id(0) == 0)
    def _():
        mx_ref[...] = jnp.full((1, 16), -jnp.inf, jnp.float32)

    cur = jnp.concatenate([jnp.max(as8, axis=0, keepdims=True),
                           jnp.max(ad8, axis=0, keepdims=True)], axis=1)
    mx_ref[...] = jnp.maximum(mx_ref[...], cur)

    @pl.when(pl.program_id(0) == NGRID - 1)
    def _():
        m = mx_ref[...]
        mm = m[:, :8] + m[:, 8:]
        mm = jnp.maximum(mm, mm * 0.2)
        mx_ref[...] = jnp.concatenate([mm, mm], axis=1)


def _tc_gat_prep(h, Wg, As, Ad):
    return pl.pallas_call(
        _tc_gat_prep_kernel,
        grid=(NGRID,),
        in_specs=[_nb(DH), _fixed((DH, HEADS * DH)),
                  _fixed((HEADS * DH, HEADS)), _fixed((HEADS * DH, HEADS))],
        out_specs=[_nb(HEADS * DH), _nb(16), _nb(16), _fixed((1, 16))],
        out_shape=[
            jax.ShapeDtypeStruct((N, HEADS * DH), jnp.float32),
            jax.ShapeDtypeStruct((N, 16), jnp.float32),
            jax.ShapeDtypeStruct((N, 16), jnp.float32),
            jax.ShapeDtypeStruct((1, 16), jnp.float32),
        ],
    )(h, Wg, As, Ad)


def _tc_gat_mid_kernel(ast_ref, adt_ref, de_ref, m_ref,
                       adit_ref, salf_ref):
    als = ast_ref[...][:, :8]
    ald = adt_ref[...][:, :8]
    l = als + ald
    l = jnp.maximum(l, l * 0.2)
    exs = jnp.exp(l - m_ref[...][:, :8])
    den = de_ref[...][:, :8] + exs
    inv = 1.0 / (den + 1e-16)
    adit_ref[...] = jnp.concatenate([ald, inv], axis=1)
    salf_ref[...] = exs * inv


def _tc_gat_mid(ASt, ADt, den_edges, M16):
    return pl.pallas_call(
        _tc_gat_mid_kernel,
        grid=(NGRID,),
        in_specs=[_nb(16), _nb(16), _nb(16), _fixed((1, 16))],
        out_specs=[_nb(16), _nb(HEADS)],
        out_shape=[
            jax.ShapeDtypeStruct((N, 16), jnp.float32),
            jax.ShapeDtypeStruct((N, HEADS), jnp.float32),
        ],
    )(ASt, ADt, den_edges, M16)


def _tc_gat_final_kernel(num_ref, salf_ref, ht_ref, bg_ref, o_ref):
    acc = num_ref[...]
    salf = salf_ref[...]
    ht = ht_ref[...]
    for h in range(HEADS):
        acc = acc + salf[:, h:h + 1] * ht[:, h * DH:(h + 1) * DH]
    o_ref[...] = acc * (1.0 / HEADS) + bg_ref[...]


def _tc_gat_final(num, SAlf, Ht, bg):
    return pl.pallas_call(
        _tc_gat_final_kernel,
        grid=(NGRID,),
        in_specs=[_nb(DH), _nb(HEADS), _nb(HEADS * DH), _fixed((1, DH))],
        out_specs=_nb(DH),
        out_shape=jax.ShapeDtypeStruct((N, DH), jnp.float32),
    )(num, SAlf, Ht, bg.reshape(1, -1))


def _final_head(p0, p1, Wf, bf):
    return pl.pallas_call(
        _head_kernel,
        out_shape=jax.ShapeDtypeStruct((G, Wf.shape[1]), jnp.float32),
    )(p0, p1, Wf, bf.reshape(1, -1))


def _merge(p):
    return jnp.concatenate([p[0, :HALF], p[1, :HALF]], axis=0)[:N]


def kernel(x, edge_index, batch, W1, b1, g1, be1, W2, b2, g2, be2, W3, b3,
           g3, be3, Wg, a_src, a_dst, bg, Wf, bf):
    pad = EPAD - E
    src2 = jnp.concatenate(
        [edge_index[0], jnp.zeros((pad,), jnp.int32)]).reshape(ER2, 128)
    dst2 = jnp.concatenate(
        [edge_index[1], jnp.full((pad,), N, jnp.int32)]).reshape(ER2, 128)

    degm = _merge(_sc_deg(dst2))
    dis = _tc_dis(degm[:, :1])

    h = x
    for W, gm, be in ((W1, g1, be1), (W2, g2, be2), (W3, g3, be3)):
        u = _tc_mm_scale(h, W, dis)
        C = W.shape[1] // 16
        uc = u.reshape(N, C, 16).transpose(1, 0, 2)
        P = jnp.concatenate(
            [_merge(_sc_seg16(uc[ci], src2, dst2)) for ci in range(C)],
            axis=1)
        st = _tc_stats(P, u, dis)
        h = _tc_bn_relu(P, u, dis, st, gm, be)

    lanes = jnp.arange(HEADS * DH)
    As = jnp.zeros((HEADS * DH, HEADS), jnp.float32).at[
        lanes, lanes // DH].set(a_src.reshape(-1))
    Ad = jnp.zeros((HEADS * DH, HEADS), jnp.float32).at[
        lanes, lanes // DH].set(a_dst.reshape(-1))
    Ht, ASt, ADt, M16 = _tc_gat_prep(h, Wg, As, Ad)
    den_edges = _merge(_sc_gat_den(ASt, ADt, M16.reshape(16), src2, dst2))
    ADIt, SAlf = _tc_gat_mid(ASt, ADt, den_edges, M16)
    num = _merge(_sc_gat_num(ASt, ADIt, Ht, M16.reshape(16), src2, dst2))
    gat = _tc_gat_final(num, SAlf, Ht, bg)

    gp = jnp.concatenate([gat, jnp.zeros((PN - N, DH), jnp.float32)])
    b2 = jnp.concatenate(
        [batch, jnp.full((PN - N,), G, jnp.int32)]).reshape(PR2, 128)
    poolp = _sc_pool(gp, b2)
    return _final_head(poolp[0], poolp[1], Wf, bf)


# speedup vs baseline: 1.2003x; 1.2003x over previous
import functools

import jax
import jax.numpy as jnp
from jax import lax
from jax.experimental import pallas as pl
from jax.experimental.pallas import tpu as pltpu
from jax.experimental.pallas import tpu_sc as plsc

N = 100000
E = 3200000
G = 256
HEADS = 8
DH = 16
EPS = 1e-5

NC = 2
NS = 16
NW = NC * NS
RPT = 1568
ER2 = NS * RPT
EPAD = ER2 * 128
KB = 16
NBLK = RPT // KB
KB2 = 8
HALF = 50048
ACC_ROWS = 50176
NPT = ACC_ROWS // NS
ZCH = 392
NZCH = NPT // ZCH
PR2 = 1024
PN = PR2 * 128
PACC_ROWS = 512
PRPT = PR2 // NW

_MESH = plsc.VectorSubcoreMesh(core_axis_name="c", subcore_axis_name="s")


def _vgather(v, idx):
    return lax.gather(
        v, idx[:, None],
        dimension_numbers=lax.GatherDimensionNumbers(
            offset_dims=(), collapsed_slice_dims=(0,), start_index_map=(0,)),
        slice_sizes=(1,),
        mode=lax.GatherScatterMode.PROMISE_IN_BOUNDS)


def _zero_acc(acc, zbuf):
    def zb(i, _):
        zbuf[i, :] = jnp.zeros((16,), jnp.float32)
        return 0
    lax.fori_loop(0, ZCH, zb, 0)
    s = lax.axis_index("s")
    for k in range(NZCH):
        pltpu.sync_copy(zbuf, acc.at[pl.ds(s * NPT + k * ZCH, ZCH)])


def _localize(didx, lidx, kb):
    cbase = lax.axis_index("c") * HALF
    lanes = lax.iota(jnp.int32, 16)

    def tr(i, _):
        for j in range(8):
            d = didx[i, j * 16:(j + 1) * 16]
            loc = d - cbase
            ok = (loc >= 0) & (loc < HALF)
            lidx[i, j * 16:(j + 1) * 16] = jnp.where(ok, loc, HALF + lanes)
        return 0

    lax.fori_loop(0, kb, tr, 0)


def _writeout(acc, out):
    s = lax.axis_index("s")
    c = lax.axis_index("c")
    pltpu.sync_copy(acc.at[pl.ds(s * NPT, NPT)],
                    out.at[c, pl.ds(s * NPT, NPT)])


@functools.partial(
    pl.kernel,
    out_type=jax.ShapeDtypeStruct((NC, ACC_ROWS, 16), jnp.float32),
    mesh=_MESH,
    compiler_params=pltpu.CompilerParams(use_tc_tiling_on_sc=False),
    scratch_types=[
        pltpu.VMEM((KB, 128), jnp.int32),
        pltpu.VMEM((KB, 128), jnp.int32),
        pltpu.VMEM((KB * 128, 16), jnp.float32),
        pltpu.VMEM((ZCH, 16), jnp.float32),
        pltpu.VMEM_SHARED((ACC_ROWS, 16), jnp.float32),
        pltpu.SemaphoreType.DMA,
    ],
)
def _sc_seg16(table, src2, dst2, out, sidx, didx, rows, zbuf, acc, sem):
    _zero_acc(acc, zbuf)
    plsc.subcore_barrier()
    base = lax.axis_index("s") * RPT

    def blk(b, _):
        rb = base + b * KB
        pltpu.sync_copy(src2.at[pl.ds(rb, KB)], sidx)
        pltpu.sync_copy(dst2.at[pl.ds(rb, KB)], didx)
        _localize(didx, didx, KB)
        descs = [
            pltpu.async_copy(table.at[sidx.at[j]],
                             rows.at[pl.ds(j * 128, 128)], sem)
            for j in range(KB)
        ]
        for d in descs:
            d.wait()
        for j in range(KB):
            pltpu.sync_copy(rows.at[pl.ds(j * 128, 128)],
                            acc.at[didx.at[j]], add=True)
        return 0

    lax.fori_loop(0, NBLK, blk, 0)
    plsc.subcore_barrier()
    _writeout(acc, out)


@functools.partial(
    pl.kernel,
    out_type=jax.ShapeDtypeStruct((NC, ACC_ROWS, 16), jnp.float32),
    mesh=_MESH,
    compiler_params=pltpu.CompilerParams(use_tc_tiling_on_sc=False),
    scratch_types=[
        pltpu.VMEM((KB, 128), jnp.int32),
        pltpu.VMEM((128, 16), jnp.float32),
        pltpu.VMEM((ZCH, 16), jnp.float32),
        pltpu.VMEM_SHARED((ACC_ROWS, 16), jnp.float32),
    ],
)
def _sc_deg(dst2, out, didx, ones, zbuf, acc):
    _zero_acc(acc, zbuf)
    def ob(i, _):
        ones[i, :] = jnp.full((16,), 1.0, jnp.float32)
        return 0
    lax.fori_loop(0, 128, ob, 0)
    plsc.subcore_barrier()
    base = lax.axis_index("s") * RPT

    def blk(b, _):
        rb = base + b * KB
        pltpu.sync_copy(dst2.at[pl.ds(rb, KB)], didx)
        _localize(didx, didx, KB)
        for j in range(KB):
            pltpu.sync_copy(ones, acc.at[didx.at[j]], add=True)
        return 0

    lax.fori_loop(0, NBLK, blk, 0)
    plsc.subcore_barrier()
    _writeout(acc, out)


@functools.partial(
    pl.kernel,
    out_type=jax.ShapeDtypeStruct((NC, ACC_ROWS, 16), jnp.float32),
    mesh=_MESH,
    compiler_params=pltpu.CompilerParams(use_tc_tiling_on_sc=False),
    scratch_types=[
        pltpu.VMEM((KB, 128), jnp.int32),
        pltpu.VMEM((KB, 128), jnp.int32),
        pltpu.VMEM((KB, 128), jnp.int32),
        pltpu.VMEM((KB * 128, 16), jnp.float32),
        pltpu.VMEM((KB * 128, 16), jnp.float32),
        pltpu.VMEM((16,), jnp.float32),
        pltpu.VMEM((ZCH, 16), jnp.float32),
        pltpu.VMEM_SHARED((ACC_ROWS, 16), jnp.float32),
        pltpu.SemaphoreType.DMA,
    ],
)
def _sc_gat_den(ast, adt, mvec, src2, dst2, out,
                sidx, didx, lidx, rs, rd, mbuf, zbuf, acc, sem):
    _zero_acc(acc, zbuf)
    pltpu.sync_copy(mvec, mbuf)
    plsc.subcore_barrier()
    base = lax.axis_index("s") * RPT

    def blk(b, _):
        rb = base + b * KB
        pltpu.sync_copy(src2.at[pl.ds(rb, KB)], sidx)
        pltpu.sync_copy(dst2.at[pl.ds(rb, KB)], didx)
        _localize(didx, lidx, KB)
        descs = [
            pltpu.async_copy(ast.at[sidx.at[j]],
                             rs.at[pl.ds(j * 128, 128)], sem)
            for j in range(KB)
        ] + [
            pltpu.async_copy(adt.at[didx.at[j]],
                             rd.at[pl.ds(j * 128, 128)], sem)
            for j in range(KB)
        ]
        for d in descs:
            d.wait()
        m = mbuf[:]

        def ed(i, _):
            for k in range(8):
                e = i * 8 + k
                l = rs[e, :] + rd[e, :]
                l = jnp.maximum(l, l * 0.2)
                rs[e, :] = jnp.exp(l - m)
            return 0

        lax.fori_loop(0, KB * 16, ed, 0)
        for j in range(KB):
            pltpu.sync_copy(rs.at[pl.ds(j * 128, 128)],
                            acc.at[lidx.at[j]], add=True)
        return 0

    lax.fori_loop(0, NBLK, blk, 0)
    plsc.subcore_barrier()
    _writeout(acc, out)


@functools.partial(
    pl.kernel,
    out_type=jax.ShapeDtypeStruct((NC, ACC_ROWS, 16), jnp.float32),
    mesh=_MESH,
    compiler_params=pltpu.CompilerParams(use_tc_tiling_on_sc=False),
    scratch_types=[
        pltpu.VMEM((KB2, 128), jnp.int32),
        pltpu.VMEM((KB2, 128), jnp.int32),
        pltpu.VMEM((KB2, 128), jnp.int32),
        pltpu.VMEM((KB2 * 128, 16), jnp.float32),
        pltpu.VMEM((KB2 * 128, 16), jnp.float32),
        pltpu.VMEM((2, 128, 128), jnp.float32),
        pltpu.VMEM((128, 16), jnp.float32),
        pltpu.VMEM((16,), jnp.float32),
        pltpu.VMEM((ZCH, 16), jnp.float32),
        pltpu.VMEM_SHARED((ACC_ROWS, 16), jnp.float32),
        pltpu.SemaphoreType.DMA,
        pltpu.SemaphoreType.DMA,
        pltpu.SemaphoreType.DMA,
    ],
)
def _sc_gat_num(ast, adit, htab, mvec, src2, dst2, out,
                sidx, didx, lidx, rs, rdi, rh, contrib, mbuf, zbuf, acc,
                sem, sem2, sem3):
    _zero_acc(acc, zbuf)
    pltpu.sync_copy(mvec, mbuf)
    plsc.subcore_barrier()
    base = lax.axis_index("s") * RPT
    perm = (lax.iota(jnp.int32, 16) % 8) + 8

    def blk(b, _):
        rb = base + b * KB2
        pltpu.sync_copy(src2.at[pl.ds(rb, KB2)], sidx)
        pltpu.sync_copy(dst2.at[pl.ds(rb, KB2)], didx)
        _localize(didx, lidx, KB2)
        ds_rs = [
            pltpu.async_copy(ast.at[sidx.at[j]],
                             rs.at[pl.ds(j * 128, 128)], sem)
            for j in range(KB2)
        ]
        ds_rdi = [
            pltpu.async_copy(adit.at[didx.at[j]],
                             rdi.at[pl.ds(j * 128, 128)], sem2)
            for j in range(KB2)
        ]
        d_rh = pltpu.async_copy(htab.at[sidx.at[0]], rh.at[0], sem3)
        for d in ds_rs:
            d.wait()
        for d in ds_rdi:
            d.wait()
        m = mbuf[:]
        for j in range(KB2):
            if j + 1 < KB2:
                d_next = pltpu.async_copy(htab.at[sidx.at[j + 1]],
                                          rh.at[(j + 1) % 2], sem3)
            d_rh.wait()
            rhj = rh.at[j % 2]

            def ed(i, _):
                for k in range(2):
                    e = i * 2 + k
                    di = rdi[j * 128 + e, :]
                    l = rs[j * 128 + e, :] + di
                    l = jnp.maximum(l, l * 0.2)
                    ex = jnp.exp(l - m)
                    alpha = ex * _vgather(di, perm)
                    acc16 = jnp.zeros((16,), jnp.float32)
                    for h in range(HEADS):
                        sp = _vgather(alpha, jnp.full((16,), h, jnp.int32))
                        acc16 = acc16 + sp * rhj[e, h * 16:(h + 1) * 16]
                    contrib[e, :] = acc16
                return 0

            lax.fori_loop(0, 64, ed, 0)
            pltpu.sync_copy(contrib, acc.at[lidx.at[j]], add=True)
            if j + 1 < KB2:
                d_rh = d_next
        return 0

    lax.fori_loop(0, RPT // KB2, blk, 0)
    plsc.subcore_barrier()
    _writeout(acc, out)


@functools.partial(
    pl.kernel,
    out_type=jax.ShapeDtypeStruct((NC, G, 16), jnp.float32),
    mesh=_MESH,
    compiler_params=pltpu.CompilerParams(use_tc_tiling_on_sc=False),
    scratch_types=[
        pltpu.VMEM((PRPT, 128), jnp.int32),
        pltpu.VMEM((128, 16), jnp.float32),
        pltpu.VMEM((PACC_ROWS // NS, 16), jnp.float32),
        pltpu.VMEM_SHARED((PACC_ROWS, 16), jnp.float32),
    ],
)
def _sc_pool(vals, b2, out, bidx, vbuf, zbuf, acc):
    s = lax.axis_index("s")
    c = lax.axis_index("c")
    def zb(i, _):
        zbuf[i, :] = jnp.zeros((16,), jnp.float32)
        return 0
    lax.fori_loop(0, PACC_ROWS // NS, zb, 0)
    pltpu.sync_copy(zbuf, acc.at[pl.ds(s * (PACC_ROWS // NS),
                                       PACC_ROWS // NS)])
    plsc.subcore_barrier()
    base = (s * NC + c) * PRPT
    pltpu.sync_copy(b2.at[pl.ds(base, PRPT)], bidx)

    def row(r, _):
        pltpu.sync_copy(vals.at[pl.ds((base + r) * 128, 128)], vbuf)
        pltpu.sync_copy(vbuf, acc.at[bidx.at[r]], add=True)
        return 0

    lax.fori_loop(0, PRPT, row, 0)
    plsc.subcore_barrier()
    pltpu.sync_copy(acc.at[pl.ds(s * (G // NS), G // NS)],
                    out.at[c, pl.ds(s * (G // NS), G // NS)])


def _head_kernel(p0_ref, p1_ref, w_ref, b_ref, o_ref):
    o_ref[...] = (
        jnp.dot(p0_ref[...] + p1_ref[...], w_ref[...],
                preferred_element_type=jnp.float32)
        + b_ref[...]
    )


BLK = 2000
NGRID = N // BLK


def _nb(d):
    return pl.BlockSpec((BLK, d), lambda i: (i, 0))


def _fixed(shape):
    return pl.BlockSpec(shape, lambda i: tuple(0 for _ in shape))


def _tc_dis_kernel(d_ref, o_ref):
    o_ref[...] = lax.rsqrt(d_ref[...] + 1.0)


def _tc_dis(d):
    return pl.pallas_call(
        _tc_dis_kernel,
        grid=(NGRID,),
        in_specs=[_nb(1)],
        out_specs=_nb(1),
        out_shape=jax.ShapeDtypeStruct((N, 1), jnp.float32),
    )(d)


def _tc_mm_scale_kernel(x_ref, w_ref, dis_ref, o_ref):
    o_ref[...] = (
        jnp.dot(x_ref[...], w_ref[...], preferred_element_type=jnp.float32)
        * dis_ref[...]
    )


def _tc_mm_scale(x, W, dis):
    din, dout = W.shape
    return pl.pallas_call(
        _tc_mm_scale_kernel,
        grid=(NGRID,),
        in_specs=[_nb(din), _fixed((din, dout)), _nb(1)],
        out_specs=_nb(dout),
        out_shape=jax.ShapeDtypeStruct((N, dout), jnp.float32),
    )(x, W, dis)


def _tc_stats_kernel(p_ref, u_ref, dis_ref, o_ref):
    y = (p_ref[...] + u_ref[...]) * dis_ref[...]

    @pl.when(pl.program_id(0) == 0)
    def _():
        o_ref[...] = jnp.zeros_like(o_ref)

    o_ref[0, :] += jnp.sum(y, axis=0)
    o_ref[1, :] += jnp.sum(y * y, axis=0)


def _tc_stats(p, u, dis):
    dout = u.shape[1]
    return pl.pallas_call(
        _tc_stats_kernel,
        grid=(NGRID,),
        in_specs=[_nb(dout), _nb(dout), _nb(1)],
        out_specs=_fixed((2, dout)),
        out_shape=jax.ShapeDtypeStruct((2, dout), jnp.float32),
    )(p, u, dis)


def _tc_bn_kernel(p_ref, u_ref, dis_ref, st_ref, g_ref, be_ref, o_ref):
    y = (p_ref[...] + u_ref[...]) * dis_ref[...]
    mu = st_ref[0, :] / N
    var = st_ref[1, :] / N - mu * mu
    xn = g_ref[...] * (y - mu[None, :]) * lax.rsqrt(var[None, :] + EPS)
    o_ref[...] = jax.nn.relu(xn + be_ref[...])


def _tc_bn_relu(p, u, dis, st, gm, be):
    dout = u.shape[1]
    return pl.pallas_call(
        _tc_bn_kernel,
        grid=(NGRID,),
        in_specs=[_nb(dout), _nb(dout), _nb(1), _fixed((2, dout)),
                  _fixed((1, dout)), _fixed((1, dout))],
        out_specs=_nb(dout),
        out_shape=jax.ShapeDtypeStruct((N, dout), jnp.float32),
    )(p, u, dis, st, gm.reshape(1, -1), be.reshape(1, -1))


def _tc_gat_prep_kernel(h_ref, wg_ref, as_ref, ad_ref,
                        ht_ref, ast_ref, adt_ref, mx_ref):
    ht = jnp.dot(h_ref[...], wg_ref[...], preferred_element_type=jnp.float32)
    as8 = jnp.dot(ht, as_ref[...], preferred_element_type=jnp.float32)
    ad8 = jnp.dot(ht, ad_ref[...], preferred_element_type=jnp.float32)
    ht_ref[...] = ht
    ast_ref[...] = jnp.concatenate([as8, as8], axis=1)
    adt_ref[...] = jnp.concatenate([ad8, ad8], axis=1)

    @pl.when(pl.program_id(0) == 0)
    def _():
        mx_ref[...] = jnp.full((1, 16), -jnp.inf, jnp.float32)

    cur = jnp.concatenate([jnp.max(as8, axis=0, keepdims=True),
                           jnp.max(ad8, axis=0, keepdims=True)], axis=1)
    mx_ref[...] = jnp.maximum(mx_ref[...], cur)

    @pl.when(pl.program_id(0) == NGRID - 1)
    def _():
        m = mx_ref[...]
        mm = m[:, :8] + m[:, 8:]
        mm = jnp.maximum(mm, mm * 0.2)
        mx_ref[...] = jnp.concatenate([mm, mm], axis=1)


def _tc_gat_prep(h, Wg, As, Ad):
    return pl.pallas_call(
        _tc_gat_prep_kernel,
        grid=(NGRID,),
        in_specs=[_nb(DH), _fixed((DH, HEADS * DH)),
                  _fixed((HEADS * DH, HEADS)), _fixed((HEADS * DH, HEADS))],
        out_specs=[_nb(HEADS * DH), _nb(16), _nb(16), _fixed((1, 16))],
        out_shape=[
            jax.ShapeDtypeStruct((N, HEADS * DH), jnp.float32),
            jax.ShapeDtypeStruct((N, 16), jnp.float32),
            jax.ShapeDtypeStruct((N, 16), jnp.float32),
            jax.ShapeDtypeStruct((1, 16), jnp.float32),
        ],
    )(h, Wg, As, Ad)


def _tc_gat_mid_kernel(ast_ref, adt_ref, de_ref, m_ref,
                       adit_ref, salf_ref):
    als = ast_ref[...][:, :8]
    ald = adt_ref[...][:, :8]
    l = als + ald
    l = jnp.maximum(l, l * 0.2)
    exs = jnp.exp(l - m_ref[...][:, :8])
    den = de_ref[...][:, :8] + exs
    inv = 1.0 / (den + 1e-16)
    adit_ref[...] = jnp.concatenate([ald, inv], axis=1)
    salf_ref[...] = exs * inv


def _tc_gat_mid(ASt, ADt, den_edges, M16):
    return pl.pallas_call(
        _tc_gat_mid_kernel,
        grid=(NGRID,),
        in_specs=[_nb(16), _nb(16), _nb(16), _fixed((1, 16))],
        out_specs=[_nb(16), _nb(HEADS)],
        out_shape=[
            jax.ShapeDtypeStruct((N, 16), jnp.float32),
            jax.ShapeDtypeStruct((N, HEADS), jnp.float32),
        ],
    )(ASt, ADt, den_edges, M16)


def _tc_gat_final_kernel(num_ref, salf_ref, ht_ref, bg_ref, o_ref):
    acc = num_ref[...]
    salf = salf_ref[...]
    ht = ht_ref[...]
    for h in range(HEADS):
        acc = acc + salf[:, h:h + 1] * ht[:, h * DH:(h + 1) * DH]
    o_ref[...] = acc * (1.0 / HEADS) + bg_ref[...]


def _tc_gat_final(num, SAlf, Ht, bg):
    return pl.pallas_call(
        _tc_gat_final_kernel,
        grid=(NGRID,),
        in_specs=[_nb(DH), _nb(HEADS), _nb(HEADS * DH), _fixed((1, DH))],
        out_specs=_nb(DH),
        out_shape=jax.ShapeDtypeStruct((N, DH), jnp.float32),
    )(num, SAlf, Ht, bg.reshape(1, -1))


def _final_head(p0, p1, Wf, bf):
    return pl.pallas_call(
        _head_kernel,
        out_shape=jax.ShapeDtypeStruct((G, Wf.shape[1]), jnp.float32),
    )(p0, p1, Wf, bf.reshape(1, -1))


def _merge(p):
    return jnp.concatenate([p[0, :HALF], p[1, :HALF]], axis=0)[:N]


def kernel(x, edge_index, batch, W1, b1, g1, be1, W2, b2, g2, be2, W3, b3,
           g3, be3, Wg, a_src, a_dst, bg, Wf, bf):
    pad = EPAD - E
    src2 = jnp.concatenate(
        [edge_index[0], jnp.zeros((pad,), jnp.int32)]).reshape(ER2, 128)
    dst2 = jnp.concatenate(
        [edge_index[1], jnp.full((pad,), N, jnp.int32)]).reshape(ER2, 128)

    degm = _merge(_sc_deg(dst2))
    dis = _tc_dis(degm[:, :1])

    h = x
    for W, gm, be in ((W1, g1, be1), (W2, g2, be2), (W3, g3, be3)):
        u = _tc_mm_scale(h, W, dis)
        C = W.shape[1] // 16
        uc = u.reshape(N, C, 16).transpose(1, 0, 2)
        P = jnp.concatenate(
            [_merge(_sc_seg16(uc[ci], src2, dst2)) for ci in range(C)],
            axis=1)
        st = _tc_stats(P, u, dis)
        h = _tc_bn_relu(P, u, dis, st, gm, be)

    lanes = jnp.arange(HEADS * DH)
    As = jnp.zeros((HEADS * DH, HEADS), jnp.float32).at[
        lanes, lanes // DH].set(a_src.reshape(-1))
    Ad = jnp.zeros((HEADS * DH, HEADS), jnp.float32).at[
        lanes, lanes // DH].set(a_dst.reshape(-1))
    Ht, ASt, ADt, M16 = _tc_gat_prep(h, Wg, As, Ad)
    den_edges = _merge(_sc_gat_den(ASt, ADt, M16.reshape(16), src2, dst2))
    ADIt, SAlf = _tc_gat_mid(ASt, ADt, den_edges, M16)
    num = _merge(_sc_gat_num(ASt, ADIt, Ht, M16.reshape(16), src2, dst2))
    gat = _tc_gat_final(num, SAlf, Ht, bg)

    gp = jnp.concatenate([gat, jnp.zeros((PN - N, DH), jnp.float32)])
    b2 = jnp.concatenate(
        [batch, jnp.full((PN - N,), G, jnp.int32)]).reshape(PR2, 128)
    poolp = _sc_pool(gp, b2)
    return _final_head(poolp[0], poolp[1], Wf, bf)
